# baseline (speedup 1.0000x reference)
import functools

import jax
import jax.numpy as jnp
from jax import lax
from jax.experimental import pallas as pl
from jax.experimental.pallas import tpu as pltpu
from jax.experimental.pallas import tpu_sc as plsc


_BS = 2048


def _add_kernel(x_ref, emb_ref, o_ref):
    o_ref[0] = x_ref[0] + emb_ref[...]


def _kernel_tc(x, emb_weight):
    B, S, D = x.shape
    emb = emb_weight[:S]
    grid = (S // _BS, B)
    return pl.pallas_call(
        _add_kernel,
        grid=grid,
        in_specs=[
            pl.BlockSpec((1, _BS, D), lambda i, j: (j, i, 0)),
            pl.BlockSpec((_BS, D), lambda i, j: (i, 0)),
        ],
        out_specs=pl.BlockSpec((1, _BS, D), lambda i, j: (j, i, 0)),
        out_shape=jax.ShapeDtypeStruct((B, S, D), x.dtype),
    )(x, emb)


_NC = 2
_NS = 16
_NW = _NC * _NS
_RP = 8
_NSLOT = 3


def _kernel_sc_pipe(x, emb_weight):
    B, S, D = x.shape
    emb = emb_weight[:S]
    seq_per_w = S // _NW
    n_chunks = seq_per_w // _RP
    n_full = (n_chunks // _NSLOT) * _NSLOT
    mesh = plsc.VectorSubcoreMesh(core_axis_name="c", subcore_axis_name="s")

    @functools.partial(
        pl.kernel,
        out_type=jax.ShapeDtypeStruct((B, S, D), jnp.float32),
        mesh=mesh,
        scratch_types=[
            pltpu.VMEM((_NSLOT, B, _RP, D), jnp.float32),
            pltpu.VMEM((_NSLOT, _RP, D), jnp.float32),
            pltpu.SemaphoreType.DMA((_NSLOT,)),
            pltpu.SemaphoreType.DMA((_NSLOT,)),
            pltpu.SemaphoreType.DMA((_NSLOT,)),
        ],
    )
    def k(x_hbm, emb_hbm, out_hbm, xb, eb, sem_in, sem_e, sem_out):
        wid = lax.axis_index("s") * _NC + lax.axis_index("c")
        seq0 = wid * seq_per_w

        def start_in(slot, c):
            s0 = seq0 + c * _RP
            pltpu.make_async_copy(
                x_hbm.at[:, pl.ds(s0, _RP)], xb.at[slot], sem_in.at[slot]
            ).start()
            pltpu.make_async_copy(
                emb_hbm.at[pl.ds(s0, _RP)], eb.at[slot], sem_e.at[slot]
            ).start()

        def wait_in(slot, c):
            s0 = seq0 + c * _RP
            pltpu.make_async_copy(
                x_hbm.at[:, pl.ds(s0, _RP)], xb.at[slot], sem_in.at[slot]
            ).wait()
            pltpu.make_async_copy(
                emb_hbm.at[pl.ds(s0, _RP)], eb.at[slot], sem_e.at[slot]
            ).wait()

        def start_out(slot, c):
            s0 = seq0 + c * _RP
            pltpu.make_async_copy(
                xb.at[slot], out_hbm.at[:, pl.ds(s0, _RP)], sem_out.at[slot]
            ).start()

        def wait_out(slot, c):
            s0 = seq0 + c * _RP
            pltpu.make_async_copy(
                xb.at[slot], out_hbm.at[:, pl.ds(s0, _RP)], sem_out.at[slot]
            ).wait()

        def compute(slot):
            def col_body(ct, _):
                sl = pl.ds(ct * 16, 16)
                for r in range(_RP):
                    e = eb[slot, r, sl]
                    for b in range(B):
                        xb[slot, b, r, sl] = xb[slot, b, r, sl] + e
                return 0

            lax.fori_loop(0, D // 16, col_body, 0)

        def process(slot, c):
            wait_in(slot, c)
            compute(slot)
            start_out(slot, c)
            ns = (slot + 2) % _NSLOT

            @pl.when(c >= _NSLOT - 2)
            def _():
                wait_out(ns, c - (_NSLOT - 2))

            @pl.when(c + 2 < n_chunks)
            def _():
                start_in(ns, c + 2)

        start_in(0, 0)
        start_in(1, 1)

        def step(t, _):
            for kk in range(_NSLOT):
                process(kk, _NSLOT * t + kk)
            return 0

        lax.fori_loop(0, n_full // _NSLOT, step, 0)
        for c in range(n_full, n_chunks):
            process(c % _NSLOT, c)
        for c in range(n_chunks - (_NSLOT - 2), n_chunks):
            wait_out(c % _NSLOT, c)

    return k(x, emb)


def kernel(x, emb_weight):
    return _kernel_sc_pipe(x, emb_weight)


# speedup vs baseline: 1.0167x; 1.0167x over previous
import functools

import jax
import jax.numpy as jnp
from jax import lax
from jax.experimental import pallas as pl
from jax.experimental.pallas import tpu as pltpu
from jax.experimental.pallas import tpu_sc as plsc


_BS = 2048


def _add_kernel(x_ref, emb_ref, o_ref):
    o_ref[0] = x_ref[0] + emb_ref[...]


def _kernel_tc(x, emb_weight):
    B, S, D = x.shape
    emb = emb_weight[:S]
    grid = (S // _BS, B)
    return pl.pallas_call(
        _add_kernel,
        grid=grid,
        in_specs=[
            pl.BlockSpec((1, _BS, D), lambda i, j: (j, i, 0)),
            pl.BlockSpec((_BS, D), lambda i, j: (i, 0)),
        ],
        out_specs=pl.BlockSpec((1, _BS, D), lambda i, j: (j, i, 0)),
        out_shape=jax.ShapeDtypeStruct((B, S, D), x.dtype),
    )(x, emb)


_NC = 2
_NS = 16
_NW = _NC * _NS
_RP = 4
_NSLOT = 4


def _kernel_sc_pipe(x, emb_weight):
    B, S, D = x.shape
    emb = emb_weight[:S]
    seq_per_w = S // _NW
    n_chunks = seq_per_w // _RP
    n_full = (n_chunks // _NSLOT) * _NSLOT
    mesh = plsc.VectorSubcoreMesh(core_axis_name="c", subcore_axis_name="s")

    @functools.partial(
        pl.kernel,
        out_type=jax.ShapeDtypeStruct((B, S, D), jnp.float32),
        mesh=mesh,
        scratch_types=[
            pltpu.VMEM((_NSLOT, B, _RP, D), jnp.float32),
            pltpu.VMEM((_NSLOT, _RP, D), jnp.float32),
            pltpu.SemaphoreType.DMA((_NSLOT,)),
            pltpu.SemaphoreType.DMA((_NSLOT,)),
            pltpu.SemaphoreType.DMA((_NSLOT,)),
        ],
    )
    def k(x_hbm, emb_hbm, out_hbm, xb, eb, sem_in, sem_e, sem_out):
        wid = lax.axis_index("s") * _NC + lax.axis_index("c")
        seq0 = wid * seq_per_w

        def start_in(slot, c):
            s0 = seq0 + c * _RP
            pltpu.make_async_copy(
                x_hbm.at[:, pl.ds(s0, _RP)], xb.at[slot], sem_in.at[slot]
            ).start()
            pltpu.make_async_copy(
                emb_hbm.at[pl.ds(s0, _RP)], eb.at[slot], sem_e.at[slot]
            ).start()

        def wait_in(slot, c):
            s0 = seq0 + c * _RP
            pltpu.make_async_copy(
                x_hbm.at[:, pl.ds(s0, _RP)], xb.at[slot], sem_in.at[slot]
            ).wait()
            pltpu.make_async_copy(
                emb_hbm.at[pl.ds(s0, _RP)], eb.at[slot], sem_e.at[slot]
            ).wait()

        def start_out(slot, c):
            s0 = seq0 + c * _RP
            pltpu.make_async_copy(
                xb.at[slot], out_hbm.at[:, pl.ds(s0, _RP)], sem_out.at[slot]
            ).start()

        def wait_out(slot, c):
            s0 = seq0 + c * _RP
            pltpu.make_async_copy(
                xb.at[slot], out_hbm.at[:, pl.ds(s0, _RP)], sem_out.at[slot]
            ).wait()

        def compute(slot):
            def col_body(ct, _):
                sl = pl.ds(ct * 16, 16)
                for r in range(_RP):
                    e = eb[slot, r, sl]
                    for b in range(B):
                        xb[slot, b, r, sl] = xb[slot, b, r, sl] + e
                return 0

            lax.fori_loop(0, D // 16, col_body, 0)

        def process(slot, c):
            wait_in(slot, c)
            compute(slot)
            start_out(slot, c)
            ns = (slot + 2) % _NSLOT

            @pl.when(c >= _NSLOT - 2)
            def _():
                wait_out(ns, c - (_NSLOT - 2))

            @pl.when(c + 2 < n_chunks)
            def _():
                start_in(ns, c + 2)

        start_in(0, 0)
        start_in(1, 1)

        def step(t, _):
            for kk in range(_NSLOT):
                process(kk, _NSLOT * t + kk)
            return 0

        lax.fori_loop(0, n_full // _NSLOT, step, 0)
        for c in range(n_full, n_chunks):
            process(c % _NSLOT, c)
        for c in range(n_chunks - (_NSLOT - 2), n_chunks):
            wait_out(c % _NSLOT, c)

    return k(x, emb)


def kernel(x, emb_weight):
    return _kernel_sc_pipe(x, emb_weight)


# speedup vs baseline: 1.0464x; 1.0293x over previous
import functools

import jax
import jax.numpy as jnp
from jax import lax
from jax.experimental import pallas as pl
from jax.experimental.pallas import tpu as pltpu
from jax.experimental.pallas import tpu_sc as plsc


_BS = 2048


def _add_kernel(x_ref, emb_ref, o_ref):
    o_ref[0] = x_ref[0] + emb_ref[...]


def _kernel_tc(x, emb_weight):
    B, S, D = x.shape
    emb = emb_weight[:S]
    grid = (S // _BS, B)
    return pl.pallas_call(
        _add_kernel,
        grid=grid,
        in_specs=[
            pl.BlockSpec((1, _BS, D), lambda i, j: (j, i, 0)),
            pl.BlockSpec((_BS, D), lambda i, j: (i, 0)),
        ],
        out_specs=pl.BlockSpec((1, _BS, D), lambda i, j: (j, i, 0)),
        out_shape=jax.ShapeDtypeStruct((B, S, D), x.dtype),
    )(x, emb)


_NC = 2
_NS = 16
_NW = _NC * _NS
_RP = 4
_NSLOT = 6
_PF = 4


def _kernel_sc_pipe(x, emb_weight):
    B, S, D = x.shape
    emb = emb_weight[:S]
    seq_per_w = S // _NW
    n_chunks = seq_per_w // _RP
    n_full = (n_chunks // _NSLOT) * _NSLOT
    mesh = plsc.VectorSubcoreMesh(core_axis_name="c", subcore_axis_name="s")

    @functools.partial(
        pl.kernel,
        out_type=jax.ShapeDtypeStruct((B, S, D), jnp.float32),
        mesh=mesh,
        scratch_types=[
            pltpu.VMEM((_NSLOT, B, _RP, D), jnp.float32),
            pltpu.VMEM((_NSLOT, _RP, D), jnp.float32),
            pltpu.SemaphoreType.DMA((_NSLOT,)),
            pltpu.SemaphoreType.DMA((_NSLOT,)),
            pltpu.SemaphoreType.DMA((_NSLOT,)),
        ],
    )
    def k(x_hbm, emb_hbm, out_hbm, xb, eb, sem_in, sem_e, sem_out):
        wid = lax.axis_index("s") * _NC + lax.axis_index("c")
        seq0 = wid * seq_per_w

        def start_in(slot, c):
            s0 = seq0 + c * _RP
            pltpu.make_async_copy(
                x_hbm.at[:, pl.ds(s0, _RP)], xb.at[slot], sem_in.at[slot]
            ).start()
            pltpu.make_async_copy(
                emb_hbm.at[pl.ds(s0, _RP)], eb.at[slot], sem_e.at[slot]
            ).start()

        def wait_in(slot, c):
            s0 = seq0 + c * _RP
            pltpu.make_async_copy(
                x_hbm.at[:, pl.ds(s0, _RP)], xb.at[slot], sem_in.at[slot]
            ).wait()
            pltpu.make_async_copy(
                emb_hbm.at[pl.ds(s0, _RP)], eb.at[slot], sem_e.at[slot]
            ).wait()

        def start_out(slot, c):
            s0 = seq0 + c * _RP
            pltpu.make_async_copy(
                xb.at[slot], out_hbm.at[:, pl.ds(s0, _RP)], sem_out.at[slot]
            ).start()

        def wait_out(slot, c):
            s0 = seq0 + c * _RP
            pltpu.make_async_copy(
                xb.at[slot], out_hbm.at[:, pl.ds(s0, _RP)], sem_out.at[slot]
            ).wait()

        def compute(slot):
            def col_body(ct, _):
                sl = pl.ds(ct * 16, 16)
                for r in range(_RP):
                    e = eb[slot, r, sl]
                    for b in range(B):
                        xb[slot, b, r, sl] = xb[slot, b, r, sl] + e
                return 0

            lax.fori_loop(0, D // 16, col_body, 0)

        def process(slot, c):
            wait_in(slot, c)
            compute(slot)
            start_out(slot, c)
            ns = (slot + _PF) % _NSLOT

            @pl.when(c >= _NSLOT - _PF)
            def _():
                wait_out(ns, c - (_NSLOT - _PF))

            @pl.when(c + _PF < n_chunks)
            def _():
                start_in(ns, c + _PF)

        for c in range(_PF):
            start_in(c, c)

        def step(t, _):
            for kk in range(_NSLOT):
                process(kk, _NSLOT * t + kk)
            return 0

        lax.fori_loop(0, n_full // _NSLOT, step, 0)
        for c in range(n_full, n_chunks):
            process(c % _NSLOT, c)
        for c in range(n_chunks - (_NSLOT - _PF), n_chunks):
            wait_out(c % _NSLOT, c)

    return k(x, emb)


def kernel(x, emb_weight):
    return _kernel_sc_pipe(x, emb_weight)


# speedup vs baseline: 1.0543x; 1.0076x over previous
import functools

import jax
import jax.numpy as jnp
from jax import lax
from jax.experimental import pallas as pl
from jax.experimental.pallas import tpu as pltpu
from jax.experimental.pallas import tpu_sc as plsc


_BS = 2048


def _add_kernel(x_ref, emb_ref, o_ref):
    o_ref[0] = x_ref[0] + emb_ref[...]


def _kernel_tc(x, emb_weight):
    B, S, D = x.shape
    emb = emb_weight[:S]
    grid = (S // _BS, B)
    return pl.pallas_call(
        _add_kernel,
        grid=grid,
        in_specs=[
            pl.BlockSpec((1, _BS, D), lambda i, j: (j, i, 0)),
            pl.BlockSpec((_BS, D), lambda i, j: (i, 0)),
        ],
        out_specs=pl.BlockSpec((1, _BS, D), lambda i, j: (j, i, 0)),
        out_shape=jax.ShapeDtypeStruct((B, S, D), x.dtype),
    )(x, emb)


_NC = 2
_NS = 16
_NW = _NC * _NS
_RP = 2
_NSLOT = 8
_PF = 6


def _kernel_sc_pipe(x, emb_weight):
    B, S, D = x.shape
    emb = emb_weight[:S]
    seq_per_w = S // _NW
    n_chunks = seq_per_w // _RP
    n_full = (n_chunks // _NSLOT) * _NSLOT
    mesh = plsc.VectorSubcoreMesh(core_axis_name="c", subcore_axis_name="s")

    @functools.partial(
        pl.kernel,
        out_type=jax.ShapeDtypeStruct((B, S, D), jnp.float32),
        mesh=mesh,
        scratch_types=[
            pltpu.VMEM((_NSLOT, B, _RP, D), jnp.float32),
            pltpu.VMEM((_NSLOT, _RP, D), jnp.float32),
            pltpu.SemaphoreType.DMA((_NSLOT,)),
            pltpu.SemaphoreType.DMA((_NSLOT,)),
            pltpu.SemaphoreType.DMA((_NSLOT,)),
        ],
    )
    def k(x_hbm, emb_hbm, out_hbm, xb, eb, sem_in, sem_e, sem_out):
        wid = lax.axis_index("s") * _NC + lax.axis_index("c")
        seq0 = wid * seq_per_w

        def start_in(slot, c):
            s0 = seq0 + c * _RP
            pltpu.make_async_copy(
                x_hbm.at[:, pl.ds(s0, _RP)], xb.at[slot], sem_in.at[slot]
            ).start()
            pltpu.make_async_copy(
                emb_hbm.at[pl.ds(s0, _RP)], eb.at[slot], sem_e.at[slot]
            ).start()

        def wait_in(slot, c):
            s0 = seq0 + c * _RP
            pltpu.make_async_copy(
                x_hbm.at[:, pl.ds(s0, _RP)], xb.at[slot], sem_in.at[slot]
            ).wait()
            pltpu.make_async_copy(
                emb_hbm.at[pl.ds(s0, _RP)], eb.at[slot], sem_e.at[slot]
            ).wait()

        def start_out(slot, c):
            s0 = seq0 + c * _RP
            pltpu.make_async_copy(
                xb.at[slot], out_hbm.at[:, pl.ds(s0, _RP)], sem_out.at[slot]
            ).start()

        def wait_out(slot, c):
            s0 = seq0 + c * _RP
            pltpu.make_async_copy(
                xb.at[slot], out_hbm.at[:, pl.ds(s0, _RP)], sem_out.at[slot]
            ).wait()

        def compute(slot):
            def col_body(ct, _):
                sl = pl.ds(ct * 16, 16)
                for r in range(_RP):
                    e = eb[slot, r, sl]
                    for b in range(B):
                        xb[slot, b, r, sl] = xb[slot, b, r, sl] + e
                return 0

            lax.fori_loop(0, D // 16, col_body, 0)

        def process(slot, c):
            wait_in(slot, c)
            compute(slot)
            start_out(slot, c)
            ns = (slot + _PF) % _NSLOT

            @pl.when(c >= _NSLOT - _PF)
            def _():
                wait_out(ns, c - (_NSLOT - _PF))

            @pl.when(c + _PF < n_chunks)
            def _():
                start_in(ns, c + _PF)

        for c in range(_PF):
            start_in(c, c)

        def step(t, _):
            for kk in range(_NSLOT):
                process(kk, _NSLOT * t + kk)
            return 0

        lax.fori_loop(0, n_full // _NSLOT, step, 0)
        for c in range(n_full, n_chunks):
            process(c % _NSLOT, c)
        for c in range(n_chunks - (_NSLOT - _PF), n_chunks):
            wait_out(c % _NSLOT, c)

    return k(x, emb)


def kernel(x, emb_weight):
    return _kernel_sc_pipe(x, emb_weight)
